# Initial kernel scaffold; baseline (speedup 1.0000x reference)
import jax
import jax.numpy as jnp
from jax.experimental import pallas as pl


def kernel(x, table):
    raise NotImplementedError("write your pallas kernel here")


# speedup vs baseline: 8.4823x; 8.4823x over previous
import functools

import jax
import jax.numpy as jnp
from jax import lax
from jax.experimental import pallas as pl
from jax.experimental.pallas import tpu as pltpu
from jax.experimental.pallas import tpu_sc as plsc

VOCAB = 100000
EMB = 128
BATCH = 4096
HIST = 200

NC = 2
NS = 16
NW = NC * NS
B = BATCH * HIST
BPW = B // NW
CH = 128
NCHUNK = BPW // CH
NG = NCHUNK // 2

_mesh = plsc.VectorSubcoreMesh(core_axis_name="c", subcore_axis_name="s")


@functools.partial(
    pl.kernel,
    out_type=jax.ShapeDtypeStruct((B, EMB), jnp.float32),
    mesh=_mesh,
    scratch_types=[
        pltpu.VMEM((NCHUNK, CH), jnp.int32),
        pltpu.VMEM((CH, EMB), jnp.float32),
        pltpu.VMEM((CH, EMB), jnp.float32),
        pltpu.SemaphoreType.DMA,
        pltpu.SemaphoreType.DMA,
        pltpu.SemaphoreType.DMA,
        pltpu.SemaphoreType.DMA,
    ],
)
def _emb_lookup(table_hbm, idx_hbm, out_hbm, idx_v, rows0, rows1,
                semg0, semg1, semw0, semw1):
    wid = lax.axis_index("s") * NC + lax.axis_index("c")
    base = wid * BPW

    pltpu.sync_copy(idx_hbm.at[wid], idx_v)

    pltpu.async_copy(table_hbm.at[idx_v.at[0]], rows0, semg0)
    pltpu.async_copy(table_hbm.at[idx_v.at[1]], rows1, semg1)

    def body(g, carry):
        a = 2 * g
        b = a + 1
        pltpu.make_async_copy(table_hbm.at[idx_v.at[a]], rows0, semg0).wait()
        pltpu.async_copy(rows0, out_hbm.at[pl.ds(base + a * CH, CH)], semw0)
        pltpu.make_async_copy(table_hbm.at[idx_v.at[b]], rows1, semg1).wait()
        pltpu.async_copy(rows1, out_hbm.at[pl.ds(base + b * CH, CH)], semw1)
        na = jnp.minimum(a + 2, NCHUNK - 1)
        nb = jnp.minimum(b + 2, NCHUNK - 1)
        pltpu.make_async_copy(rows0, out_hbm.at[pl.ds(base, CH)], semw0).wait()
        pltpu.async_copy(table_hbm.at[idx_v.at[na]], rows0, semg0)
        pltpu.make_async_copy(rows1, out_hbm.at[pl.ds(base, CH)], semw1).wait()
        pltpu.async_copy(table_hbm.at[idx_v.at[nb]], rows1, semg1)
        return carry

    lax.fori_loop(0, NG, body, 0)

    pltpu.make_async_copy(table_hbm.at[idx_v.at[0]], rows0, semg0).wait()
    pltpu.make_async_copy(table_hbm.at[idx_v.at[1]], rows1, semg1).wait()


def kernel(x, table):
    idx = x.astype(jnp.int32).reshape(NW, NCHUNK, CH)
    out = _emb_lookup(table.astype(jnp.float32), idx)
    return out.reshape(BATCH, HIST, EMB)


# speedup vs baseline: 9.0129x; 1.0626x over previous
import functools

import jax
import jax.numpy as jnp
from jax import lax
from jax.experimental import pallas as pl
from jax.experimental.pallas import tpu as pltpu
from jax.experimental.pallas import tpu_sc as plsc

VOCAB = 100000
EMB = 128
BATCH = 4096
HIST = 200

NC = 2
NS = 16
NW = NC * NS
B = BATCH * HIST
BPW = B // NW
CH = 128
NCHUNK = BPW // CH
NBUF = 4
NG = NCHUNK // NBUF

_mesh = plsc.VectorSubcoreMesh(core_axis_name="c", subcore_axis_name="s")


@functools.partial(
    pl.kernel,
    out_type=jax.ShapeDtypeStruct((B, EMB), jnp.float32),
    mesh=_mesh,
    scratch_types=[
        pltpu.VMEM((NCHUNK, CH), jnp.int32),
        [pltpu.VMEM((CH, EMB), jnp.float32)] * NBUF,
        [pltpu.SemaphoreType.DMA] * NBUF,
        [pltpu.SemaphoreType.DMA] * NBUF,
    ],
)
def _emb_lookup(table_hbm, idx_hbm, out_hbm, idx_v, rows, semg, semw):
    wid = lax.axis_index("s") * NC + lax.axis_index("c")
    base = wid * BPW

    pltpu.sync_copy(idx_hbm.at[wid], idx_v)

    for k in range(NBUF):
        pltpu.async_copy(table_hbm.at[idx_v.at[k]], rows[k], semg[k])

    def body(g, carry):
        c0 = NBUF * g
        for k in range(NBUF):
            c = c0 + k
            pltpu.make_async_copy(table_hbm.at[idx_v.at[c]], rows[k],
                                  semg[k]).wait()
            pltpu.async_copy(rows[k], out_hbm.at[pl.ds(base + c * CH, CH)],
                             semw[k])
        for k in range(NBUF):
            nc = jnp.minimum(c0 + k + NBUF, NCHUNK - 1)
            pltpu.make_async_copy(rows[k], out_hbm.at[pl.ds(base, CH)],
                                  semw[k]).wait()
            pltpu.async_copy(table_hbm.at[idx_v.at[nc]], rows[k], semg[k])
        return carry

    lax.fori_loop(0, NG, body, 0)

    for k in range(NBUF):
        pltpu.make_async_copy(table_hbm.at[idx_v.at[0]], rows[k],
                              semg[k]).wait()


def kernel(x, table):
    idx = x.astype(jnp.int32).reshape(NW, NCHUNK, CH)
    out = _emb_lookup(table.astype(jnp.float32), idx)
    return out.reshape(BATCH, HIST, EMB)
